# Initial kernel scaffold; baseline (speedup 1.0000x reference)
import functools

import jax
import jax.numpy as jnp
from jax import lax
from jax.experimental import pallas as pl
from jax.experimental.pallas import tpu as pltpu
from jax.experimental.pallas import tpu_sc as plsc

D_TOKEN = 64
BATCH = 4096
HIST = 200
VOCAB = 100

NUM_CORES = 2
NUM_SUBCORES = 16
NW = NUM_CORES * NUM_SUBCORES
TOTAL = BATCH * HIST
PER_W = TOTAL // NW
CHUNK = 128
N_CHUNKS = PER_W // CHUNK


@functools.partial(
    pl.kernel,
    out_type=jax.ShapeDtypeStruct((NW, N_CHUNKS, CHUNK, D_TOKEN), jnp.float32),
    mesh=plsc.VectorSubcoreMesh(core_axis_name="c", subcore_axis_name="s"),
    scratch_types=[
        pltpu.VMEM((N_CHUNKS, CHUNK), jnp.int32),
        pltpu.VMEM((CHUNK, D_TOKEN), jnp.float32),
        pltpu.SemaphoreType.DMA,
        pltpu.SemaphoreType.DMA,
    ],
)
def _sc_gather(table_hbm, idx_hbm, out_hbm, idx_v, rows_v, gsem, osem):
    wid = lax.axis_index("s") * NUM_CORES + lax.axis_index("c")
    pltpu.sync_copy(idx_hbm.at[wid], idx_v)

    def step(j, _):
        pltpu.async_copy(table_hbm.at[idx_v.at[j]], rows_v, gsem).wait()
        pltpu.async_copy(rows_v, out_hbm.at[wid].at[j], osem).wait()
        return ()

    lax.fori_loop(0, N_CHUNKS, step, ())


def kernel(x, dic):
    table = jnp.concatenate([dic[-1:], dic[:-1]], axis=0)
    idx = x.reshape(NW, N_CHUNKS, CHUNK)
    out = _sc_gather(table, idx)
    return out.reshape(BATCH, HIST, D_TOKEN)


# speedup vs baseline: 2.2856x; 2.2856x over previous
import functools

import jax
import jax.numpy as jnp
from jax import lax
from jax.experimental import pallas as pl
from jax.experimental.pallas import tpu as pltpu
from jax.experimental.pallas import tpu_sc as plsc

D_TOKEN = 64
BATCH = 4096
HIST = 200
VOCAB = 100

NUM_CORES = 2
NUM_SUBCORES = 16
NW = NUM_CORES * NUM_SUBCORES
TOTAL = BATCH * HIST
PER_W = TOTAL // NW
CHUNK = 128
N_CHUNKS = PER_W // CHUNK


@functools.partial(
    pl.kernel,
    out_type=jax.ShapeDtypeStruct((NW, N_CHUNKS, CHUNK, D_TOKEN), jnp.float32),
    mesh=plsc.VectorSubcoreMesh(core_axis_name="c", subcore_axis_name="s"),
    compiler_params=pltpu.CompilerParams(use_tc_tiling_on_sc=False),
    scratch_types=[
        pltpu.VMEM((N_CHUNKS, CHUNK), jnp.int32),
        pltpu.VMEM((CHUNK, D_TOKEN), jnp.float32),
        pltpu.SemaphoreType.DMA,
        pltpu.SemaphoreType.DMA,
    ],
)
def _sc_gather(table_hbm, idx_hbm, out_hbm, idx_v, rows_v, gsem, osem):
    wid = lax.axis_index("s") * NUM_CORES + lax.axis_index("c")
    pltpu.sync_copy(idx_hbm.at[wid], idx_v)

    def step(j, _):
        pltpu.async_copy(table_hbm.at[idx_v.at[j]], rows_v, gsem).wait()
        pltpu.async_copy(rows_v, out_hbm.at[wid].at[j], osem).wait()
        return ()

    lax.fori_loop(0, N_CHUNKS, step, ())


def kernel(x, dic):
    table = jnp.concatenate([dic[-1:], dic[:-1]], axis=0)
    idx = x.reshape(NW, N_CHUNKS, CHUNK)
    out = _sc_gather(table, idx)
    return out.reshape(BATCH, HIST, D_TOKEN)


# speedup vs baseline: 2.2957x; 1.0044x over previous
import functools

import jax
import jax.numpy as jnp
from jax import lax
from jax.experimental import pallas as pl
from jax.experimental.pallas import tpu as pltpu
from jax.experimental.pallas import tpu_sc as plsc

D_TOKEN = 64
BATCH = 4096
HIST = 200
VOCAB = 100

NUM_CORES = 2
NUM_SUBCORES = 16
NW = NUM_CORES * NUM_SUBCORES
TOTAL = BATCH * HIST
PER_W = TOTAL // NW
CHUNK = 128
K = 5
GROUP = K * CHUNK
N_GROUPS = PER_W // GROUP


@functools.partial(
    pl.kernel,
    out_type=jax.ShapeDtypeStruct((NW, N_GROUPS, GROUP, D_TOKEN), jnp.float32),
    mesh=plsc.VectorSubcoreMesh(core_axis_name="c", subcore_axis_name="s"),
    compiler_params=pltpu.CompilerParams(use_tc_tiling_on_sc=False),
    scratch_types=[
        pltpu.VMEM((N_GROUPS, K, CHUNK), jnp.int32),
        pltpu.VMEM((2, GROUP, D_TOKEN), jnp.float32),
        pltpu.SemaphoreType.DMA((2,)),
        pltpu.SemaphoreType.DMA((2,)),
    ],
)
def _sc_gather(table_hbm, idx_hbm, out_hbm, idx_v, rows_v, gsem, osem):
    wid = lax.axis_index("s") * NUM_CORES + lax.axis_index("c")
    pltpu.sync_copy(idx_hbm.at[wid], idx_v)

    def fire_gathers(g, s):
        for k in range(K):
            pltpu.async_copy(
                table_hbm.at[idx_v.at[g].at[k]],
                rows_v.at[s].at[pl.ds(k * CHUNK, CHUNK)],
                gsem.at[s],
            )

    def drain_gathers(g, s):
        pltpu.make_async_copy(out_hbm.at[wid].at[g], rows_v.at[s], gsem.at[s]).wait()

    fire_gathers(0, 0)
    fire_gathers(1, 1)

    def body(g0, _):
        for s in range(2):
            g = g0 + s
            drain_gathers(g, s)
            pltpu.async_copy(rows_v.at[s], out_hbm.at[wid].at[g], osem.at[s])

            @pl.when(g + 2 < N_GROUPS)
            def _():
                pltpu.make_async_copy(
                    rows_v.at[s], out_hbm.at[wid].at[g], osem.at[s]
                ).wait()
                fire_gathers(g + 2, s)

        return ()

    lax.fori_loop(0, N_GROUPS // 2, lambda p, c: body(2 * p, c), ())

    for s in range(2):
        pltpu.make_async_copy(
            rows_v.at[s], out_hbm.at[wid].at[N_GROUPS - 2 + s], osem.at[s]
        ).wait()


def kernel(x, dic):
    table = jnp.concatenate([dic[-1:], dic[:-1]], axis=0)
    idx = x.reshape(NW, N_GROUPS, K, CHUNK)
    out = _sc_gather(table, idx)
    return out.reshape(BATCH, HIST, D_TOKEN)


# speedup vs baseline: 4.1115x; 1.7909x over previous
import functools

import jax
import jax.numpy as jnp
from jax import lax
from jax.experimental import pallas as pl
from jax.experimental.pallas import tpu as pltpu
from jax.experimental.pallas import tpu_sc as plsc

D_TOKEN = 64
BATCH = 4096
HIST = 200
VOCAB = 100

NUM_CORES = 2
NUM_SUBCORES = 16
NW = NUM_CORES * NUM_SUBCORES
BCOL = BATCH // NW
N_PAIRS = HIST // 2
LANES = 16
NBG = BCOL // LANES


@functools.partial(
    pl.kernel,
    out_type=jax.ShapeDtypeStruct((HIST, D_TOKEN, BATCH), jnp.float32),
    mesh=plsc.VectorSubcoreMesh(core_axis_name="c", subcore_axis_name="s"),
    compiler_params=pltpu.CompilerParams(
        use_tc_tiling_on_sc=True, needs_layout_passes=False
    ),
    scratch_types=[
        pltpu.VMEM((VOCAB * D_TOKEN,), jnp.float32),
        pltpu.VMEM((8, BCOL), jnp.int32),
        pltpu.VMEM((2, D_TOKEN, BCOL), jnp.float32),
        pltpu.SemaphoreType.DMA((2,)),
    ],
)
def _sc_emb(table_hbm, xt_hbm, out_hbm, tab_v, idx_v, p_v, osem):
    wid = lax.axis_index("s") * NUM_CORES + lax.axis_index("c")
    col = wid * BCOL
    pltpu.sync_copy(table_hbm, tab_v)

    def compute_block(r, slot):
        for bg in range(NBG):
            iv = idx_v[r, pl.ds(bg * LANES, LANES)]
            base = iv * D_TOKEN

            @plsc.parallel_loop(0, D_TOKEN, unroll=8)
            def _(d):
                p_v[slot, d, pl.ds(bg * LANES, LANES)] = plsc.load_gather(
                    tab_v, [base + d]
                )

    def pair_body(gp, _):
        @pl.when(gp % 4 == 0)
        def _():
            pltpu.sync_copy(
                xt_hbm.at[pl.ds((gp // 4) * 8, 8), pl.ds(col, BCOL)], idx_v
            )

        for k in range(2):
            h = gp * 2 + k
            r = (gp % 4) * 2 + k

            @pl.when(gp > 0)
            def _():
                pltpu.make_async_copy(
                    p_v.at[k], out_hbm.at[h].at[:, pl.ds(col, BCOL)], osem.at[k]
                ).wait()

            compute_block(r, k)
            pltpu.async_copy(
                p_v.at[k], out_hbm.at[h].at[:, pl.ds(col, BCOL)], osem.at[k]
            )

        return ()

    lax.fori_loop(0, N_PAIRS, pair_body, ())

    for k in range(2):
        pltpu.make_async_copy(
            p_v.at[k], out_hbm.at[HIST - 2 + k].at[:, pl.ds(col, BCOL)], osem.at[k]
        ).wait()


def kernel(x, dic):
    table = jnp.concatenate([dic[-1:], dic[:-1]], axis=0).reshape(VOCAB * D_TOKEN)
    p = _sc_emb(table, x.T)
    return p.transpose(2, 0, 1)


# speedup vs baseline: 21.7033x; 5.2787x over previous
import functools

import jax
import jax.numpy as jnp
from jax import lax
from jax.experimental import pallas as pl
from jax.experimental.pallas import tpu as pltpu
from jax.experimental.pallas import tpu_sc as plsc

D_TOKEN = 64
BATCH = 4096
HIST = 200
VOCAB = 100

NUM_CORES = 2
NUM_SUBCORES = 16
NW = NUM_CORES * NUM_SUBCORES
BCOL = BATCH // NW
N_PAIRS = HIST // 2
LANES = 16
NBG = BCOL // LANES
TROW = D_TOKEN + 1


@functools.partial(
    pl.kernel,
    out_type=jax.ShapeDtypeStruct((HIST, D_TOKEN, BATCH), jnp.float32),
    mesh=plsc.VectorSubcoreMesh(core_axis_name="c", subcore_axis_name="s"),
    compiler_params=pltpu.CompilerParams(
        use_tc_tiling_on_sc=True, needs_layout_passes=False
    ),
    scratch_types=[
        pltpu.VMEM((VOCAB * TROW,), jnp.float32),
        pltpu.VMEM((8, BCOL), jnp.int32),
        pltpu.VMEM((2, D_TOKEN, BCOL), jnp.float32),
        pltpu.SemaphoreType.DMA((2,)),
    ],
)
def _sc_emb(table_hbm, xt_hbm, out_hbm, tab_v, idx_v, p_v, osem):
    wid = lax.axis_index("s") * NUM_CORES + lax.axis_index("c")
    col = wid * BCOL
    pltpu.sync_copy(table_hbm, tab_v)

    def compute_block(r, slot):
        for bg in range(NBG):
            iv = idx_v[r, pl.ds(bg * LANES, LANES)]
            base = iv * TROW

            @plsc.parallel_loop(0, D_TOKEN, unroll=8)
            def _(d):
                p_v[slot, d, pl.ds(bg * LANES, LANES)] = plsc.load_gather(
                    tab_v, [base + d]
                )

    def pair_body(gp, _):
        @pl.when(gp % 4 == 0)
        def _():
            pltpu.sync_copy(
                xt_hbm.at[pl.ds((gp // 4) * 8, 8), pl.ds(col, BCOL)], idx_v
            )

        for k in range(2):
            h = gp * 2 + k
            r = (gp % 4) * 2 + k

            @pl.when(gp > 0)
            def _():
                pltpu.make_async_copy(
                    p_v.at[k], out_hbm.at[h].at[:, pl.ds(col, BCOL)], osem.at[k]
                ).wait()

            compute_block(r, k)
            pltpu.async_copy(
                p_v.at[k], out_hbm.at[h].at[:, pl.ds(col, BCOL)], osem.at[k]
            )

        return ()

    lax.fori_loop(0, N_PAIRS, pair_body, ())

    for k in range(2):
        pltpu.make_async_copy(
            p_v.at[k], out_hbm.at[HIST - 2 + k].at[:, pl.ds(col, BCOL)], osem.at[k]
        ).wait()


def kernel(x, dic):
    table = jnp.concatenate([dic[-1:], dic[:-1]], axis=0)
    table = jnp.pad(table, ((0, 0), (0, TROW - D_TOKEN))).reshape(VOCAB * TROW)
    p = _sc_emb(table, x.T)
    return p.transpose(2, 0, 1)
